# baseline (speedup 1.0000x reference)
import functools

import jax
import jax.numpy as jnp
from jax import lax
from jax.experimental import pallas as pl
from jax.experimental.pallas import tpu as pltpu
from jax.experimental.pallas import tpu_sc as plsc

R, C = 16384, 512
OC = C // 2
NW = 32
ROWS_PER_W = R // NW
N_CHUNK = 16
CH_ROWS = ROWS_PER_W // N_CHUNK
LANES = 16
VECS_PER_ROW = OC // LANES
N_IN_BUF = 3
N_OUT_BUF = 2

_mesh = plsc.VectorSubcoreMesh(core_axis_name="c", subcore_axis_name="s")


@functools.partial(
    pl.kernel,
    mesh=_mesh,
    out_type=jax.ShapeDtypeStruct((R, OC), jnp.float32),
    scratch_types=[
        *[pltpu.VMEM((CH_ROWS, C), jnp.float32) for _ in range(N_IN_BUF)],
        *[pltpu.VMEM((CH_ROWS, OC), jnp.float32) for _ in range(N_OUT_BUF)],
        pltpu.SemaphoreType.DMA,
        pltpu.SemaphoreType.DMA,
    ],
    compiler_params=pltpu.CompilerParams(needs_layout_passes=False),
)
def _deinterleave(in_hbm, out_hbm, in_v0, in_v1, in_v2, out_v0, out_v1,
                  in_sem, out_sem):
    wid = lax.axis_index("s") * 2 + lax.axis_index("c")
    row_base = wid * ROWS_PER_W
    iota2 = lax.iota(jnp.int32, LANES) * 2
    cols = [iota2 + j * (2 * LANES) for j in range(VECS_PER_ROW)]
    in_bufs = (in_v0, in_v1, in_v2)
    out_bufs = (out_v0, out_v1)

    def in_copy(c):
        return pltpu.async_copy(
            in_hbm.at[pl.ds(row_base + c * CH_ROWS, CH_ROWS), :],
            in_bufs[c % N_IN_BUF], in_sem)

    def out_copy(c):
        return pltpu.async_copy(
            out_bufs[c % N_OUT_BUF],
            out_hbm.at[pl.ds(row_base + c * CH_ROWS, CH_ROWS), :],
            out_sem)

    in_h = [in_copy(c) for c in range(N_IN_BUF)]
    out_h = [None] * N_OUT_BUF
    for c in range(N_CHUNK):
        in_h[c % N_IN_BUF].wait()
        if out_h[c % N_OUT_BUF] is not None:
            out_h[c % N_OUT_BUF].wait()
        iv = in_bufs[c % N_IN_BUF]
        ov = out_bufs[c % N_OUT_BUF]

        @plsc.parallel_loop(0, CH_ROWS, 1, unroll=2)
        def _(r):
            row = jnp.full((LANES,), r, jnp.int32)
            for j in range(VECS_PER_ROW):
                ov[r, pl.ds(j * LANES, LANES)] = plsc.load_gather(
                    iv, [row, cols[j]])

        out_h[c % N_OUT_BUF] = out_copy(c)
        if c + N_IN_BUF < N_CHUNK:
            in_h[c % N_IN_BUF] = in_copy(c + N_IN_BUF)
    for h in out_h:
        h.wait()


def kernel(inputs):
    return _deinterleave(inputs)


# speedup vs baseline: 1.0765x; 1.0765x over previous
import functools

import jax
import jax.numpy as jnp
from jax import lax
from jax.experimental import pallas as pl
from jax.experimental.pallas import tpu as pltpu
from jax.experimental.pallas import tpu_sc as plsc

R, C = 16384, 512
OC = C // 2
NW = 32
ROWS_PER_W = R // NW
N_CHUNK = 8
CH_ROWS = ROWS_PER_W // N_CHUNK
LANES = 16
VECS_PER_ROW = OC // LANES

_mesh = plsc.VectorSubcoreMesh(core_axis_name="c", subcore_axis_name="s")


@functools.partial(
    pl.kernel,
    mesh=_mesh,
    out_type=jax.ShapeDtypeStruct((R, OC), jnp.float32),
    scratch_types=[
        pltpu.VMEM((CH_ROWS, C), jnp.float32),
        pltpu.VMEM((CH_ROWS, C), jnp.float32),
        pltpu.VMEM((CH_ROWS, OC), jnp.float32),
        pltpu.VMEM((CH_ROWS, OC), jnp.float32),
        pltpu.SemaphoreType.DMA,
        pltpu.SemaphoreType.DMA,
    ],
    compiler_params=pltpu.CompilerParams(needs_layout_passes=False),
)
def _deinterleave(in_hbm, out_hbm, in_v0, in_v1, out_v0, out_v1,
                  in_sem, out_sem):
    wid = lax.axis_index("s") * 2 + lax.axis_index("c")
    row_base = wid * ROWS_PER_W
    iota2 = lax.iota(jnp.int32, LANES) * 2
    in_bufs = (in_v0, in_v1)
    out_bufs = (out_v0, out_v1)

    def in_copy(c):
        return pltpu.async_copy(
            in_hbm.at[pl.ds(row_base + c * CH_ROWS, CH_ROWS), :],
            in_bufs[c % 2], in_sem)

    def out_copy(c):
        return pltpu.async_copy(
            out_bufs[c % 2],
            out_hbm.at[pl.ds(row_base + c * CH_ROWS, CH_ROWS), :],
            out_sem)

    in_h = in_copy(0)
    out_h = [None, None]
    for c in range(N_CHUNK):
        in_h.wait()
        if c + 1 < N_CHUNK:
            in_h = in_copy(c + 1)
        if out_h[c % 2] is not None:
            out_h[c % 2].wait()
        iv = in_bufs[c % 2]
        ov = out_bufs[c % 2]

        @plsc.parallel_loop(0, CH_ROWS * VECS_PER_ROW, 1, unroll=16)
        def _(i):
            r = i >> 4
            j = i & (VECS_PER_ROW - 1)
            col = iota2 + j * (2 * LANES)
            row = jnp.full((LANES,), r, jnp.int32)
            ov[r, pl.ds(j * LANES, LANES)] = plsc.load_gather(iv, [row, col])

        out_h[c % 2] = out_copy(c)
    out_h[0].wait()
    out_h[1].wait()


def kernel(inputs):
    return _deinterleave(inputs)
